# baseline (speedup 1.0000x reference)
import functools

import jax
import jax.numpy as jnp
from jax import lax
from jax.experimental import pallas as pl
from jax.experimental.pallas import tpu as pltpu
from jax.experimental.pallas import tpu_sc as plsc

DOC_LEN = 16384
DIMS = 128
CLASSES = 1000

NC = 2
NS = 16
NW = NC * NS
PER_W = DOC_LEN // NW
LANES = 16
NVREG = DIMS // LANES
NG = 7
HD = DIMS // 2
TAILC = CLASSES - NG * 128
SEGS = ((0, 128), (128, 128), (256, 128), (384, 112), (496, 16))


def _sc_body(nums_hbm, emb_hbm, w_hbm, out_hbm, doc_hbm,
             idx_v, rows_v, acc_v, stage_v, doc_v, w_v, log_v, shared,
             sem0, sem1, sem2, sem3, sem4, semw):
    c = lax.axis_index("c")
    s = lax.axis_index("s")
    wid = s * NC + c
    g = lax.rem(s, NG)
    half = jnp.minimum(lax.div(s, NG), 1)
    rlo = half * HD
    cw = pltpu.async_copy(
        w_hbm.at[pl.ds(rlo, HD), pl.ds(g * 128, 128)], w_v, semw)
    base = wid * PER_W
    sems = (sem0, sem1, sem2, sem3, sem4)
    first = SEGS[0][1]
    pltpu.sync_copy(nums_hbm.at[pl.ds(base, first)],
                    idx_v.at[pl.ds(0, first)])
    cps = [pltpu.async_copy(
        emb_hbm.at[idx_v.at[pl.ds(0, first)]],
        rows_v.at[pl.ds(0, first)], sems[0])]
    pltpu.sync_copy(nums_hbm.at[pl.ds(base + first, PER_W - first)],
                    idx_v.at[pl.ds(first, PER_W - first)])
    cps += [
        pltpu.async_copy(
            emb_hbm.at[idx_v.at[pl.ds(off, n)]],
            rows_v.at[pl.ds(off, n)], sems[i])
        for i, (off, n) in enumerate(SEGS) if i > 0
    ]
    accs = tuple(jnp.zeros((LANES,), jnp.float32) for _ in range(NVREG))
    for i, (off, n) in enumerate(SEGS):
        cps[i].wait()

        def row_body(r, a):
            return tuple(
                a[j] + rows_v[r, pl.ds(j * LANES, LANES)]
                for j in range(NVREG))

        accs = lax.fori_loop(off, off + n, row_body, accs)
    for j in range(NVREG):
        acc_v[pl.ds(j * LANES, LANES)] = accs[j]
    pltpu.sync_copy(acc_v, shared.at[s])
    plsc.subcore_barrier()

    @pl.when(s == 0)
    def _():
        pltpu.sync_copy(shared.at[pl.ds(0, NS)], stage_v)

        def red_body(i, a):
            return tuple(
                a[j] + stage_v[i, pl.ds(j * LANES, LANES)]
                for j in range(NVREG))

        docs = lax.fori_loop(
            0, NS, red_body,
            tuple(jnp.zeros((LANES,), jnp.float32) for _ in range(NVREG)))
        for j in range(NVREG):
            doc_v[pl.ds(j * LANES, LANES)] = docs[j]
        pltpu.sync_copy(doc_v, shared.at[NS])
        pltpu.sync_copy(doc_v, doc_hbm.at[c])

    plsc.subcore_barrier()
    pltpu.sync_copy(shared.at[NS], doc_v)
    cw.wait()
    logs = tuple(jnp.zeros((LANES,), jnp.float32) for _ in range(NVREG))

    def mv_body(j, a):
        vec = doc_v[pl.ds(rlo + j * LANES, LANES)]
        for i in range(LANES):
            dd = vec[i]
            a = tuple(
                a[q] + dd * w_v[j * LANES + i, pl.ds(q * LANES, LANES)]
                for q in range(NVREG))
        return a

    logs = lax.fori_loop(0, HD // LANES, mv_body, logs)
    for q in range(NVREG):
        log_v[pl.ds(q * LANES, LANES)] = logs[q]
    pltpu.sync_copy(log_v, out_hbm.at[c, s])


_sc_gather_mv = functools.partial(
    pl.kernel,
    mesh=plsc.VectorSubcoreMesh(core_axis_name="c", subcore_axis_name="s"),
    out_type=[
        jax.ShapeDtypeStruct((NC, NS, 128), jnp.float32),
        jax.ShapeDtypeStruct((NC, DIMS), jnp.float32),
    ],
    scratch_types=[
        pltpu.VMEM((PER_W,), jnp.int32),
        pltpu.VMEM((PER_W, DIMS), jnp.float32),
        pltpu.VMEM((DIMS,), jnp.float32),
        pltpu.VMEM((NS, DIMS), jnp.float32),
        pltpu.VMEM((DIMS,), jnp.float32),
        pltpu.VMEM((HD, 128), jnp.float32),
        pltpu.VMEM((128,), jnp.float32),
        pltpu.VMEM_SHARED((NS + 1, DIMS), jnp.float32),
        pltpu.SemaphoreType.DMA,
        pltpu.SemaphoreType.DMA,
        pltpu.SemaphoreType.DMA,
        pltpu.SemaphoreType.DMA,
        pltpu.SemaphoreType.DMA,
        pltpu.SemaphoreType.DMA,
    ],
)(_sc_body)


def _tc_body(k_ref, part_ref, doc_ref, wt_ref, b_ref, out_ref):
    p = part_ref[0] + part_ref[1]
    front = p[:NG] + p[NG:2 * NG]
    doc = doc_ref[0] + doc_ref[1]
    last = jnp.dot(doc, wt_ref[...],
                   preferred_element_type=jnp.float32)
    last = jnp.pad(last, ((0, 0), (0, 128 - TAILC)))
    raw = jnp.concatenate([front, last], axis=0) + b_ref[...]
    row = lax.broadcasted_iota(jnp.int32, (NG + 1, 128), 0)
    col = lax.broadcasted_iota(jnp.int32, (NG + 1, 128), 1)
    cls = row * 128 + col
    valid = cls < CLASSES
    logits = jnp.where(valid, raw, -1e30)
    m = jnp.max(logits)
    lse = jnp.log(jnp.sum(jnp.exp(logits - m))) + m
    sel = jnp.sum(jnp.where(jnp.logical_and(valid, cls == k_ref[0]),
                            raw, 0.0))
    out_ref[0, 0] = lse - sel


def _tc_tail(karr, partials, doc2, w_tail, b2):
    return pl.pallas_call(
        _tc_body,
        out_shape=jax.ShapeDtypeStruct((1, 1), jnp.float32),
        in_specs=[
            pl.BlockSpec(memory_space=pltpu.SMEM),
            pl.BlockSpec(memory_space=pltpu.VMEM),
            pl.BlockSpec(memory_space=pltpu.VMEM),
            pl.BlockSpec(memory_space=pltpu.VMEM),
            pl.BlockSpec(memory_space=pltpu.VMEM),
        ],
        out_specs=pl.BlockSpec(memory_space=pltpu.SMEM),
    )(karr, partials, doc2, w_tail, b2)


def kernel(nums, emb, W, b, k):
    partials, doc2 = _sc_gather_mv(nums, emb, W)
    w_tail = W[:, NG * 128:]
    b2 = jnp.pad(b, (0, (NG + 1) * 128 - CLASSES)).reshape(NG + 1, 128)
    karr = jnp.asarray(k, jnp.int32).reshape(1)
    loss = _tc_tail(karr, partials, doc2.reshape(NC, 1, DIMS), w_tail, b2)
    return loss[0, 0]


# speedup vs baseline: 1.0698x; 1.0698x over previous
import functools

import jax
import jax.numpy as jnp
from jax import lax
from jax.experimental import pallas as pl
from jax.experimental.pallas import tpu as pltpu
from jax.experimental.pallas import tpu_sc as plsc

DOC_LEN = 16384
DIMS = 128
CLASSES = 1000

NC = 2
NS = 16
NW = NC * NS
PER_W = DOC_LEN // NW
CHUNK = 128
NCHUNK = PER_W // CHUNK
LANES = 16
NVREG = DIMS // LANES
PAD_C = 1024


def _sc_body(nums_hbm, emb_hbm, out_hbm, idx_v, rows_v, acc_v, sem_a, sem_b):
    wid = lax.axis_index("s") * NC + lax.axis_index("c")
    pltpu.sync_copy(nums_hbm.at[wid], idx_v)
    sems = (sem_a, sem_b)
    cp = pltpu.async_copy(emb_hbm.at[idx_v.at[0]], rows_v.at[0], sems[0])
    accs = tuple(jnp.zeros((LANES,), jnp.float32) for _ in range(NVREG))
    for ch in range(NCHUNK):
        nxt = None
        if ch + 1 < NCHUNK:
            nxt = pltpu.async_copy(
                emb_hbm.at[idx_v.at[ch + 1]], rows_v.at[ch + 1],
                sems[(ch + 1) % 2])
        cp.wait()

        def row_body(i, a, _ch=ch):
            return tuple(
                a[j] + rows_v[_ch, i, pl.ds(j * LANES, LANES)]
                for j in range(NVREG))

        accs = lax.fori_loop(0, CHUNK, row_body, accs)
        cp = nxt
    for j in range(NVREG):
        acc_v[pl.ds(j * LANES, LANES)] = accs[j]
    pltpu.sync_copy(acc_v, out_hbm.at[wid])


_sc_gather_sum = functools.partial(
    pl.kernel,
    mesh=plsc.VectorSubcoreMesh(core_axis_name="c", subcore_axis_name="s"),
    out_type=jax.ShapeDtypeStruct((NW, DIMS), jnp.float32),
    scratch_types=[
        pltpu.VMEM((NCHUNK, CHUNK), jnp.int32),
        pltpu.VMEM((NCHUNK, CHUNK, DIMS), jnp.float32),
        pltpu.VMEM((DIMS,), jnp.float32),
        pltpu.SemaphoreType.DMA,
        pltpu.SemaphoreType.DMA,
    ],
)(_sc_body)


def _tc_body(k_ref, part_ref, w_ref, b_ref, out_ref):
    doc = jnp.sum(part_ref[...], axis=0, keepdims=True)
    logits = jnp.dot(doc, w_ref[...],
                     preferred_element_type=jnp.float32) + b_ref[...]
    m = jnp.max(logits)
    lse = jnp.log(jnp.sum(jnp.exp(logits - m))) + m
    col = lax.broadcasted_iota(jnp.int32, (1, PAD_C), 1)
    sel = jnp.sum(jnp.where(col == k_ref[0], logits, 0.0))
    out_ref[0, 0] = lse - sel


def _tc_tail(karr, partials, w_pad, b_pad):
    return pl.pallas_call(
        _tc_body,
        out_shape=jax.ShapeDtypeStruct((1, 1), jnp.float32),
        in_specs=[
            pl.BlockSpec(memory_space=pltpu.SMEM),
            pl.BlockSpec(memory_space=pltpu.VMEM),
            pl.BlockSpec(memory_space=pltpu.VMEM),
            pl.BlockSpec(memory_space=pltpu.VMEM),
        ],
        out_specs=pl.BlockSpec(memory_space=pltpu.SMEM),
    )(karr, partials, w_pad, b_pad)


def kernel(nums, emb, W, b, k):
    nums3 = nums.reshape(NW, NCHUNK, CHUNK).astype(jnp.int32)
    partials = _sc_gather_sum(nums3, emb)
    c = W.shape[1]
    w_pad = jnp.pad(W, ((0, 0), (0, PAD_C - c)))
    b_pad = jnp.pad(b, (0, PAD_C - c),
                    constant_values=-1e30).reshape(1, PAD_C)
    karr = jnp.asarray(k, jnp.int32).reshape(1)
    loss = _tc_tail(karr, partials, w_pad, b_pad)
    return loss[0, 0]
